# baseline (speedup 1.0000x reference)
import jax
import jax.numpy as jnp
from jax.experimental import pallas as pl
from jax.experimental.pallas import tpu as pltpu

N = 10000
NFEAT = 256
NCLASS = 64
BM = 400
NBUF = 3
NSTEP = N // BM


def _support_body(x_ref, w_ref, out_ref):
    out_ref[...] = jnp.dot(
        x_ref[...], w_ref[...], preferred_element_type=jnp.float32
    )


def _main_body(adj_ref, support_ref, b_ref, out_ref, buf_ref, sem_ref):
    def start_copy(i, slot):
        pltpu.make_async_copy(
            adj_ref.at[pl.ds(i * BM, BM), :], buf_ref.at[slot], sem_ref.at[slot]
        ).start()

    for i in range(NBUF):
        start_copy(i, i)

    for i in range(NSTEP):
        slot = i % NBUF
        pltpu.make_async_copy(
            adj_ref.at[pl.ds(i * BM, BM), :], buf_ref.at[slot], sem_ref.at[slot]
        ).wait()
        out = (
            jnp.dot(buf_ref[slot], support_ref[...], preferred_element_type=jnp.float32)
            + b_ref[...]
        )
        shifted = out - jnp.max(out, axis=1, keepdims=True)
        lse = jnp.log(jnp.sum(jnp.exp(shifted), axis=1, keepdims=True))
        out_ref[pl.ds(i * BM, BM), :] = shifted - lse
        if i + NBUF < NSTEP:
            start_copy(i + NBUF, slot)


@jax.jit
def kernel(x, adj, W, b):
    support = pl.pallas_call(
        _support_body,
        out_shape=jax.ShapeDtypeStruct((N, NCLASS), jnp.float32),
    )(x, W)
    b2 = b.reshape(1, NCLASS)
    return pl.pallas_call(
        _main_body,
        in_specs=[
            pl.BlockSpec(memory_space=pltpu.MemorySpace.HBM),
            pl.BlockSpec(memory_space=pltpu.MemorySpace.VMEM),
            pl.BlockSpec(memory_space=pltpu.MemorySpace.VMEM),
        ],
        out_specs=pl.BlockSpec(memory_space=pltpu.MemorySpace.VMEM),
        out_shape=jax.ShapeDtypeStruct((N, NCLASS), jnp.float32),
        scratch_shapes=[
            pltpu.VMEM((NBUF, BM, N), jnp.float32),
            pltpu.SemaphoreType.DMA((NBUF,)),
        ],
    )(adj, support, b2)


# speedup vs baseline: 1.0728x; 1.0728x over previous
import jax
import jax.numpy as jnp
from jax.experimental import pallas as pl
from jax.experimental.pallas import tpu as pltpu

N = 10000
NFEAT = 256
NCLASS = 64
BM = 400


def _body(x_ref, adj_ref, w_ref, b_ref, out_ref, support_ref):
    @pl.when(pl.program_id(0) == 0)
    def _():
        support_ref[...] = jnp.dot(
            x_ref[...], w_ref[...], preferred_element_type=jnp.float32
        )

    out = (
        jnp.dot(adj_ref[...], support_ref[...], preferred_element_type=jnp.float32)
        + b_ref[...]
    )
    shifted = out - jnp.max(out, axis=1, keepdims=True)
    lse = jnp.log(jnp.sum(jnp.exp(shifted), axis=1, keepdims=True))
    out_ref[...] = shifted - lse


@jax.jit
def kernel(x, adj, W, b):
    b2 = b.reshape(1, NCLASS)
    return pl.pallas_call(
        _body,
        grid=(pl.cdiv(N, BM),),
        in_specs=[
            pl.BlockSpec((N, NFEAT), lambda i: (0, 0)),
            pl.BlockSpec((BM, N), lambda i: (i, 0)),
            pl.BlockSpec((NFEAT, NCLASS), lambda i: (0, 0)),
            pl.BlockSpec((1, NCLASS), lambda i: (0, 0)),
        ],
        out_specs=pl.BlockSpec((BM, NCLASS), lambda i: (i, 0)),
        out_shape=jax.ShapeDtypeStruct((N, NCLASS), jnp.float32),
        scratch_shapes=[pltpu.VMEM((N, NCLASS), jnp.float32)],
    )(x, adj, W, b2)
